# Initial kernel scaffold; baseline (speedup 1.0000x reference)
import jax
import jax.numpy as jnp
from jax.experimental import pallas as pl


def kernel(encoder_out, decoder_out, targets, input_lengths, target_lengths):
    raise NotImplementedError("write your pallas kernel here")


# speedup vs baseline: 19.5699x; 19.5699x over previous
import functools
import jax
import jax.numpy as jnp
from jax import lax
from jax.experimental import pallas as pl
from jax.experimental.pallas import tpu as pltpu
from jax.experimental.pallas import tpu_sc as plsc

N = 8
TMAX = 512
UMAX = 64
V = 128
W = UMAX + 1
NCHUNK = 5
ABUF = 96
NEG = -1e30
NDIAG = TMAX - 1 + UMAX


def _log1p_poly(z):
    w = z / (z + 2.0)
    w2 = w * w
    p = w2 * (1.0 / 9.0) + (1.0 / 7.0)
    p = p * w2 + (1.0 / 5.0)
    p = p * w2 + (1.0 / 3.0)
    p = p * w2 + 1.0
    return 2.0 * (w * p)


def _body(enc_hbm, dec_hbm, y_hbm, il_hbm, tl_hbm, out_hbm,
          enc_v, dec_v, y_v, il_v, tl_v, yh_t, dh_t, db_t, a0, a1, out_v):
    cid = lax.axis_index("c")
    sid = lax.axis_index("s")
    wid = sid * 2 + cid

    @pl.when(wid < N)
    def _run():
        b = wid
        pltpu.sync_copy(enc_hbm.at[b], enc_v)
        pltpu.sync_copy(dec_hbm.at[b], dec_v)
        pltpu.sync_copy(y_hbm.at[b], y_v)
        pltpu.sync_copy(il_hbm, il_v.at[pl.ds(0, N)])
        pltpu.sync_copy(tl_hbm, tl_v.at[pl.ds(0, N)])

        iot = lax.iota(jnp.int32, 16)
        zero16 = jnp.zeros((16,), jnp.int32)
        neg16 = jnp.full((16,), NEG, jnp.float32)

        b16 = jnp.full((16,), b, jnp.int32)
        t_len = jnp.max(plsc.load_gather(il_v, [b16]))
        u_len = jnp.max(plsc.load_gather(tl_v, [b16]))
        d_target = t_len - 1 + u_len
        fin = (plsc.load_gather(enc_v, [jnp.full((16,), t_len - 1, jnp.int32), zero16])
               + plsc.load_gather(dec_v, [jnp.full((16,), u_len, jnp.int32), zero16]))

        for c in range(NCHUNK):
            u = iot + (16 * c)
            um1 = jnp.clip(u - 1, 0, UMAX - 1)
            yh = plsc.load_gather(y_v, [um1])
            yh_t[pl.ds(16 * c, 16)] = yh
            dh_t[pl.ds(16 * c, 16)] = plsc.load_gather(dec_v, [um1, yh])
            ucl = jnp.minimum(u, UMAX)
            db_t[pl.ds(16 * c, 16)] = plsc.load_gather(dec_v, [ucl, zero16])

        for c in range(ABUF // 16):
            a0[pl.ds(16 * c, 16)] = neg16
            a1[pl.ds(16 * c, 16)] = neg16
        a0[pl.ds(0, 16)] = jnp.where(iot == 1, 0.0, NEG)

        def step(d, ap, an):
            for c in range(NCHUNK):
                base = 16 * c

                @pl.when((d >= base) & (d <= 511 + base + 15))
                def _chunk():
                    u = iot + base
                    t = d - u
                    tm1 = jnp.clip(t - 1, 0, TMAX - 1)
                    tcl = jnp.clip(t, 0, TMAX - 1)
                    g_vb = plsc.load_gather(enc_v, [tm1, zero16])
                    g_he = plsc.load_gather(enc_v, [tcl, yh_t[pl.ds(base, 16)]])
                    vert = ap[pl.ds(base + 1, 16)] + g_vb + db_t[pl.ds(base, 16)]
                    horiz = ap[pl.ds(base, 16)] + g_he + dh_t[pl.ds(base, 16)]
                    m = jnp.maximum(vert, horiz)
                    s = jnp.minimum(vert, horiz)
                    z = jnp.exp(s - m)
                    an[pl.ds(base + 1, 16)] = m + _log1p_poly(z)

            @pl.when(d == d_target)
            def _capture():
                slot = jnp.full((16,), u_len + 1, jnp.int32)
                av = plsc.load_gather(an, [slot])
                out_v[pl.ds(0, 16)] = av + fin

        def loop_body(i, carry):
            d = 2 * i + 1
            step(d, a0, a1)
            step(d + 1, a1, a0)
            return carry

        lax.fori_loop(0, (NDIAG - 1) // 2, loop_body, jnp.int32(0))
        step(NDIAG, a0, a1)

        pltpu.sync_copy(out_v, out_hbm.at[b])


@jax.jit
def _rnnt_sc(enc, dec, y, il, tl):
    mesh = plsc.VectorSubcoreMesh(core_axis_name="c", subcore_axis_name="s",
                                  num_cores=2, num_subcores=16)
    f = pl.kernel(
        _body,
        out_type=jax.ShapeDtypeStruct((N, 16), jnp.float32),
        mesh=mesh,
        compiler_params=pltpu.CompilerParams(needs_layout_passes=False),
        scratch_types=[
            pltpu.VMEM((TMAX, V), jnp.float32),
            pltpu.VMEM((UMAX + 1, V), jnp.float32),
            pltpu.VMEM((UMAX,), jnp.int32),
            pltpu.VMEM((16,), jnp.int32),
            pltpu.VMEM((16,), jnp.int32),
            pltpu.VMEM((16 * NCHUNK,), jnp.int32),
            pltpu.VMEM((16 * NCHUNK,), jnp.float32),
            pltpu.VMEM((16 * NCHUNK,), jnp.float32),
            pltpu.VMEM((ABUF,), jnp.float32),
            pltpu.VMEM((ABUF,), jnp.float32),
            pltpu.VMEM((16,), jnp.float32),
        ],
    )
    return f(enc, dec, y, il, tl)


def kernel(encoder_out, decoder_out, targets, input_lengths, target_lengths):
    y = targets.astype(jnp.int32)
    il = input_lengths.astype(jnp.int32)
    tl = target_lengths.astype(jnp.int32)
    out = _rnnt_sc(encoder_out, decoder_out, y, il, tl)
    return out[:, 0]


# speedup vs baseline: 22.9118x; 1.1708x over previous
import functools
import jax
import jax.numpy as jnp
from jax import lax
from jax.experimental import pallas as pl
from jax.experimental.pallas import tpu as pltpu
from jax.experimental.pallas import tpu_sc as plsc

N = 8
TMAX = 512
UMAX = 64
V = 128
W = UMAX + 1
NCHUNK = 5
ABUF = 96
NEG = -1e30
NDIAG = TMAX - 1 + UMAX


def _log1p_poly(z):
    p = z * (-0.0172077992) + 0.0817256453
    p = p * z + (-0.188780824)
    p = p * z + 0.314589174
    p = p * z + (-0.496977431)
    p = p * z + 0.999792362
    p = p * z + 3.51102136e-06
    return p


def _body(enc_hbm, dec_hbm, y_hbm, il_hbm, tl_hbm, out_hbm,
          enc_v, dec_v, y_v, il_v, tl_v, a0, a1, out_v):
    cid = lax.axis_index("c")
    sid = lax.axis_index("s")
    wid = sid * 2 + cid

    @pl.when(wid < N)
    def _run():
        b = wid
        pltpu.sync_copy(enc_hbm.at[b], enc_v)
        pltpu.sync_copy(dec_hbm.at[b], dec_v)
        pltpu.sync_copy(y_hbm.at[b], y_v)
        pltpu.sync_copy(il_hbm, il_v.at[pl.ds(0, N)])
        pltpu.sync_copy(tl_hbm, tl_v.at[pl.ds(0, N)])

        iot = lax.iota(jnp.int32, 16)
        zero16 = jnp.zeros((16,), jnp.int32)
        neg16 = jnp.full((16,), NEG, jnp.float32)

        b16 = jnp.full((16,), b, jnp.int32)
        t_len = jnp.max(plsc.load_gather(il_v, [b16]))
        u_len = jnp.max(plsc.load_gather(tl_v, [b16]))
        d_target = t_len - 1 + u_len
        fin = (plsc.load_gather(enc_v, [jnp.full((16,), t_len - 1, jnp.int32), zero16])
               + plsc.load_gather(dec_v, [jnp.full((16,), u_len, jnp.int32), zero16]))

        consts = []
        for c in range(NCHUNK):
            u = iot + (16 * c)
            um1 = jnp.clip(u - 1, 0, UMAX - 1)
            yh = plsc.load_gather(y_v, [um1])
            dh = plsc.load_gather(dec_v, [um1, yh])
            ucl = jnp.minimum(u, UMAX)
            db = plsc.load_gather(dec_v, [ucl, zero16])
            consts.extend((u, yh, dh, db))
        consts = tuple(consts)

        for c in range(ABUF // 16):
            a0[pl.ds(16 * c, 16)] = neg16
            a1[pl.ds(16 * c, 16)] = neg16
        a0[pl.ds(0, 16)] = jnp.where(iot == 1, 0.0, NEG)

        def step(d, ap, an, cs):
            for c in range(NCHUNK):
                base = 16 * c
                u, yh, dh, db = cs[4 * c:4 * c + 4]

                @pl.when((d >= base) & (d <= 511 + base + 15))
                def _chunk():
                    t = d - u
                    tm1 = jnp.clip(t - 1, 0, TMAX - 1)
                    tcl = jnp.clip(t, 0, TMAX - 1)
                    g_vb = plsc.load_gather(enc_v, [tm1, zero16])
                    g_he = plsc.load_gather(enc_v, [tcl, yh])
                    vert = ap[pl.ds(base + 1, 16)] + g_vb + db
                    horiz = ap[pl.ds(base, 16)] + g_he + dh
                    m = jnp.maximum(vert, horiz)
                    s = jnp.minimum(vert, horiz)
                    z = jnp.exp(s - m)
                    an[pl.ds(base + 1, 16)] = m + _log1p_poly(z)

            @pl.when(d == d_target)
            def _capture():
                slot = jnp.full((16,), u_len + 1, jnp.int32)
                av = plsc.load_gather(an, [slot])
                out_v[pl.ds(0, 16)] = av + fin

        def loop_body(i, cs):
            d = 2 * i + 1
            step(d, a0, a1, cs)
            step(d + 1, a1, a0, cs)
            return cs

        consts = lax.fori_loop(0, (NDIAG - 1) // 2, loop_body, consts)
        step(NDIAG, a0, a1, consts)

        pltpu.sync_copy(out_v, out_hbm.at[b])


@jax.jit
def _rnnt_sc(enc, dec, y, il, tl):
    mesh = plsc.VectorSubcoreMesh(core_axis_name="c", subcore_axis_name="s",
                                  num_cores=2, num_subcores=16)
    f = pl.kernel(
        _body,
        out_type=jax.ShapeDtypeStruct((N, 16), jnp.float32),
        mesh=mesh,
        compiler_params=pltpu.CompilerParams(needs_layout_passes=False),
        scratch_types=[
            pltpu.VMEM((TMAX, V), jnp.float32),
            pltpu.VMEM((UMAX + 1, V), jnp.float32),
            pltpu.VMEM((UMAX,), jnp.int32),
            pltpu.VMEM((16,), jnp.int32),
            pltpu.VMEM((16,), jnp.int32),
            pltpu.VMEM((ABUF,), jnp.float32),
            pltpu.VMEM((ABUF,), jnp.float32),
            pltpu.VMEM((16,), jnp.float32),
        ],
    )
    return f(enc, dec, y, il, tl)


def kernel(encoder_out, decoder_out, targets, input_lengths, target_lengths):
    y = targets.astype(jnp.int32)
    il = input_lengths.astype(jnp.int32)
    tl = target_lengths.astype(jnp.int32)
    out = _rnnt_sc(encoder_out, decoder_out, y, il, tl)
    return out[:, 0]


# speedup vs baseline: 26.6548x; 1.1634x over previous
import jax
import jax.numpy as jnp
from jax import lax
from jax.experimental import pallas as pl
from jax.experimental.pallas import tpu as pltpu
from jax.experimental.pallas import tpu_sc as plsc

N = 8
TMAX = 512
UMAX = 64
V = 128
NCHUNK = 5
ABUF = 96
NEG = -1e30
NDIAG = TMAX - 1 + UMAX
EBP = 656


def _log1p_poly(z):
    p = z * (-0.0172077992) + 0.0817256453
    p = p * z + (-0.188780824)
    p = p * z + 0.314589174
    p = p * z + (-0.496977431)
    p = p * z + 0.999792362
    p = p * z + 3.51102136e-06
    return p


def _body(enc_hbm, dec_hbm, y_hbm, il_hbm, tl_hbm, out_hbm,
          enc_v, dec_v, y_v, il_v, tl_v, ebp, a0, a1, out_v):
    cid = lax.axis_index("c")
    sid = lax.axis_index("s")
    wid = sid * 2 + cid

    @pl.when(wid < N)
    def _run():
        b = wid
        pltpu.sync_copy(enc_hbm.at[b], enc_v)
        pltpu.sync_copy(dec_hbm.at[b], dec_v)
        pltpu.sync_copy(y_hbm.at[b], y_v)
        pltpu.sync_copy(il_hbm, il_v.at[pl.ds(0, N)])
        pltpu.sync_copy(tl_hbm, tl_v.at[pl.ds(0, N)])

        iot = lax.iota(jnp.int32, 16)
        zero16 = jnp.zeros((16,), jnp.int32)
        zf16 = jnp.zeros((16,), jnp.float32)
        neg16 = jnp.full((16,), NEG, jnp.float32)

        b16 = jnp.full((16,), b, jnp.int32)
        t_len = jnp.max(plsc.load_gather(il_v, [b16]))
        u_len = jnp.max(plsc.load_gather(tl_v, [b16]))
        d_target = t_len - 1 + u_len
        dtv = jnp.full((16,), d_target, jnp.int32)
        slot_v = jnp.full((16,), u_len + 1, jnp.int32)
        fin = (plsc.load_gather(enc_v, [jnp.full((16,), t_len - 1, jnp.int32), zero16])
               + plsc.load_gather(dec_v, [jnp.full((16,), u_len, jnp.int32), zero16]))

        for k in range(EBP // 16):
            ebp[pl.ds(16 * k, 16)] = zf16
        for k in range(TMAX // 16):
            t1 = iot + 16 * k
            vals = plsc.load_gather(enc_v, [t1, zero16])
            plsc.store_scatter(ebp, [jnp.full((16,), 574, jnp.int32) - t1], vals)

        consts = []
        for c in range(NCHUNK):
            u = iot + (16 * c)
            um1 = jnp.clip(u - 1, 0, UMAX - 1)
            yh = plsc.load_gather(y_v, [um1])
            dh = plsc.load_gather(dec_v, [um1, yh])
            ucl = jnp.minimum(u, UMAX)
            db = plsc.load_gather(dec_v, [ucl, zero16])
            consts.extend((u, yh, dh, db))
        consts = tuple(consts)

        for c in range(ABUF // 16):
            a0[pl.ds(16 * c, 16)] = neg16
            a1[pl.ds(16 * c, 16)] = neg16
        a0[pl.ds(0, 16)] = jnp.where(iot == 1, 0.0, NEG)

        def step(d, ap, an, cs):
            s0 = 575 - d
            for c in range(NCHUNK):
                base = 16 * c
                u, yh, dh, db = cs[4 * c:4 * c + 4]
                t = d - u
                tcl = jnp.clip(t, 0, TMAX - 1)
                g_he = plsc.load_gather(enc_v, [tcl, yh])
                g_vb = ebp[pl.ds(s0 + base, 16)]
                vert = ap[pl.ds(base + 1, 16)] + g_vb + db
                horiz = ap[pl.ds(base, 16)] + g_he + dh
                m = jnp.maximum(vert, horiz)
                s = jnp.minimum(vert, horiz)
                z = jnp.exp(s - m)
                an[pl.ds(base + 1, 16)] = m + _log1p_poly(z)

            hit = jnp.full((16,), d, jnp.int32) == dtv
            av = plsc.load_gather(an, [slot_v])
            out_v[pl.ds(0, 16)] = jnp.where(hit, av + fin, out_v[pl.ds(0, 16)])

        def loop_body(i, cs):
            d = 2 * i + 1
            step(d, a0, a1, cs)
            step(d + 1, a1, a0, cs)
            return cs

        consts = lax.fori_loop(0, (NDIAG - 1) // 2, loop_body, consts)
        step(NDIAG, a0, a1, consts)

        pltpu.sync_copy(out_v, out_hbm.at[b])


@jax.jit
def _rnnt_sc(enc, dec, y, il, tl):
    mesh = plsc.VectorSubcoreMesh(core_axis_name="c", subcore_axis_name="s",
                                  num_cores=2, num_subcores=16)
    f = pl.kernel(
        _body,
        out_type=jax.ShapeDtypeStruct((N, 16), jnp.float32),
        mesh=mesh,
        compiler_params=pltpu.CompilerParams(needs_layout_passes=False),
        scratch_types=[
            pltpu.VMEM((TMAX, V), jnp.float32),
            pltpu.VMEM((UMAX + 1, V), jnp.float32),
            pltpu.VMEM((UMAX,), jnp.int32),
            pltpu.VMEM((16,), jnp.int32),
            pltpu.VMEM((16,), jnp.int32),
            pltpu.VMEM((EBP,), jnp.float32),
            pltpu.VMEM((ABUF,), jnp.float32),
            pltpu.VMEM((ABUF,), jnp.float32),
            pltpu.VMEM((16,), jnp.float32),
        ],
    )
    return f(enc, dec, y, il, tl)


def kernel(encoder_out, decoder_out, targets, input_lengths, target_lengths):
    y = targets.astype(jnp.int32)
    il = input_lengths.astype(jnp.int32)
    tl = target_lengths.astype(jnp.int32)
    out = _rnnt_sc(encoder_out, decoder_out, y, il, tl)
    return out[:, 0]


# speedup vs baseline: 27.1414x; 1.0183x over previous
import jax
import jax.numpy as jnp
from jax import lax
from jax.experimental import pallas as pl
from jax.experimental.pallas import tpu as pltpu
from jax.experimental.pallas import tpu_sc as plsc

N = 8
TMAX = 512
UMAX = 64
V = 128
NCHUNK = 5
ABUF = 80
NEG = -1e30
NDIAG = TMAX - 1 + UMAX
EBP = 656
LUT = 272


def _log1p_poly(z):
    p = z * (-0.0172077992) + 0.0817256453
    p = p * z + (-0.188780824)
    p = p * z + 0.314589174
    p = p * z + (-0.496977431)
    p = p * z + 0.999792362
    p = p * z + 3.51102136e-06
    return p


_TAKE_DN = lax.GatherDimensionNumbers(
    offset_dims=(), collapsed_slice_dims=(0,), start_index_map=(0,))


def _take(v, idx):
    return lax.gather(v, idx[:, None], _TAKE_DN, slice_sizes=(1,),
                      mode=lax.GatherScatterMode.PROMISE_IN_BOUNDS)


def _body(enc_hbm, dec_hbm, y_hbm, il_hbm, tl_hbm, out_hbm,
          enc_v, dec_v, y_v, il_v, tl_v, ebp, t0_v, t1_v, ab, out_v):
    cid = lax.axis_index("c")
    sid = lax.axis_index("s")
    wid = sid * 2 + cid

    @pl.when(wid < N)
    def _run():
        b = wid
        pltpu.sync_copy(enc_hbm.at[b], enc_v)
        pltpu.sync_copy(dec_hbm.at[b], dec_v)
        pltpu.sync_copy(y_hbm.at[b], y_v)
        pltpu.sync_copy(il_hbm, il_v.at[pl.ds(0, N)])
        pltpu.sync_copy(tl_hbm, tl_v.at[pl.ds(0, N)])

        iot = lax.iota(jnp.int32, 16)
        zero16 = jnp.zeros((16,), jnp.int32)
        zf16 = jnp.zeros((16,), jnp.float32)
        neg16 = jnp.full((16,), NEG, jnp.float32)

        b16 = jnp.full((16,), b, jnp.int32)
        t_len = jnp.max(plsc.load_gather(il_v, [b16]))
        u_len = jnp.max(plsc.load_gather(tl_v, [b16]))
        d_target = t_len - 1 + u_len
        dtv = jnp.full((16,), d_target, jnp.int32)
        slot_v = jnp.full((16,), u_len, jnp.int32)
        fin = (plsc.load_gather(enc_v, [jnp.full((16,), t_len - 1, jnp.int32), zero16])
               + plsc.load_gather(dec_v, [jnp.full((16,), u_len, jnp.int32), zero16]))

        for k in range(EBP // 16):
            ebp[pl.ds(16 * k, 16)] = zf16
        for k in range(TMAX // 16):
            t1 = iot + 16 * k
            vals = plsc.load_gather(enc_v, [t1, zero16])
            plsc.store_scatter(ebp, [jnp.full((16,), 574, jnp.int32) - t1], vals)

        for k in range(LUT // 16):
            zk = jnp.minimum((iot + 16 * k).astype(jnp.float32) * (1.0 / 256.0), 1.0)
            t0_v[pl.ds(16 * k, 16)] = _log1p_poly(zk)
        for k in range(LUT // 16 - 1):
            lo = t0_v[pl.ds(16 * k, 16)]
            hi = t0_v[pl.ds(16 * k + 1, 16)]
            t1_v[pl.ds(16 * k, 16)] = hi - lo
        t1_v[pl.ds(LUT - 16, 16)] = zf16

        consts = []
        for c in range(NCHUNK):
            u = iot + (16 * c)
            um1 = jnp.clip(u - 1, 0, UMAX - 1)
            yh = plsc.load_gather(y_v, [um1])
            dh = plsc.load_gather(dec_v, [um1, yh])
            ucl = jnp.minimum(u, UMAX)
            db = plsc.load_gather(dec_v, [ucl, zero16])
            consts.extend((u, yh, dh, db))
        consts = tuple(consts)

        pm1 = jnp.maximum(iot - 1, 0)
        lane0 = iot == 0
        p15 = jnp.full((16,), 15, jnp.int32)

        r_init = [jnp.where(iot == 0, 0.0, NEG)] + [neg16] * (NCHUNK - 1)

        def step(d, rp, cs):
            s0 = 575 - d
            rn = []
            for c in range(NCHUNK):
                base = 16 * c
                u, yh, dh, db = cs[4 * c:4 * c + 4]
                t = d - u
                tcl = jnp.clip(t, 0, TMAX - 1)
                g_he = plsc.load_gather(enc_v, [tcl, yh])
                g_vb = ebp[pl.ds(s0 + base, 16)]
                sh = _take(rp[c], pm1)
                if c == 0:
                    hp = jnp.where(lane0, NEG, sh)
                else:
                    hp = jnp.where(lane0, _take(rp[c - 1], p15), sh)
                vert = rp[c] + g_vb + db
                horiz = hp + g_he + dh
                m = jnp.maximum(vert, horiz)
                s = jnp.minimum(vert, horiz)
                z = jnp.exp(s - m)
                x = z * 256.0
                xi = x.astype(jnp.int32)
                fr = x - xi.astype(jnp.float32)
                g0 = plsc.load_gather(t0_v, [xi])
                g1 = plsc.load_gather(t1_v, [xi])
                r = m + (g0 + g1 * fr)
                ab[pl.ds(base, 16)] = r
                rn.append(r)

            hit = jnp.full((16,), d, jnp.int32) == dtv
            av = plsc.load_gather(ab, [slot_v])
            out_v[pl.ds(0, 16)] = jnp.where(hit, av + fin, out_v[pl.ds(0, 16)])
            return rn

        def loop_body(i, carry):
            cs = carry[:4 * NCHUNK]
            rp = list(carry[4 * NCHUNK:])
            d = 2 * i + 1
            rp = step(d, rp, cs)
            rp = step(d + 1, rp, cs)
            return cs + tuple(rp)

        carry = lax.fori_loop(0, (NDIAG - 1) // 2, loop_body,
                              consts + tuple(r_init))
        step(NDIAG, list(carry[4 * NCHUNK:]), consts)

        pltpu.sync_copy(out_v, out_hbm.at[b])


@jax.jit
def _rnnt_sc(enc, dec, y, il, tl):
    mesh = plsc.VectorSubcoreMesh(core_axis_name="c", subcore_axis_name="s",
                                  num_cores=2, num_subcores=16)
    f = pl.kernel(
        _body,
        out_type=jax.ShapeDtypeStruct((N, 16), jnp.float32),
        mesh=mesh,
        compiler_params=pltpu.CompilerParams(needs_layout_passes=False),
        scratch_types=[
            pltpu.VMEM((TMAX, V), jnp.float32),
            pltpu.VMEM((UMAX + 1, V), jnp.float32),
            pltpu.VMEM((UMAX,), jnp.int32),
            pltpu.VMEM((16,), jnp.int32),
            pltpu.VMEM((16,), jnp.int32),
            pltpu.VMEM((EBP,), jnp.float32),
            pltpu.VMEM((LUT,), jnp.float32),
            pltpu.VMEM((LUT,), jnp.float32),
            pltpu.VMEM((ABUF,), jnp.float32),
            pltpu.VMEM((16,), jnp.float32),
        ],
    )
    return f(enc, dec, y, il, tl)


def kernel(encoder_out, decoder_out, targets, input_lengths, target_lengths):
    y = targets.astype(jnp.int32)
    il = input_lengths.astype(jnp.int32)
    tl = target_lengths.astype(jnp.int32)
    out = _rnnt_sc(encoder_out, decoder_out, y, il, tl)
    return out[:, 0]


# speedup vs baseline: 59.6562x; 2.1980x over previous
import jax
import jax.numpy as jnp
from jax import lax
from jax.experimental import pallas as pl
from jax.experimental.pallas import tpu as pltpu
from jax.experimental.pallas import tpu_sc as plsc

N = 8
TMAX = 512
UMAX = 64
V = 128
NCHUNK = 5
NEG = -1e30
NDIAG = TMAX - 1 + UMAX
EBP = 656
LUT = 1040
AH = 80 * (NDIAG + 1)


def _log1p_poly(z):
    p = z * (-0.0172077992) + 0.0817256453
    p = p * z + (-0.188780824)
    p = p * z + 0.314589174
    p = p * z + (-0.496977431)
    p = p * z + 0.999792362
    p = p * z + 3.51102136e-06
    return p


_TAKE_DN = lax.GatherDimensionNumbers(
    offset_dims=(), collapsed_slice_dims=(0,), start_index_map=(0,))


def _take(v, idx):
    return lax.gather(v, idx[:, None], _TAKE_DN, slice_sizes=(1,),
                      mode=lax.GatherScatterMode.PROMISE_IN_BOUNDS)


def _body(enc_hbm, dec_hbm, y_hbm, il_hbm, tl_hbm, out_hbm,
          enc_v, dec_v, y_v, il_v, tl_v, ebp, t0_v, t1_v, ah, out_v):
    cid = lax.axis_index("c")
    sid = lax.axis_index("s")
    wid = sid * 2 + cid

    @pl.when(wid < N)
    def _run():
        b = wid
        pltpu.sync_copy(enc_hbm.at[b], enc_v)
        pltpu.sync_copy(dec_hbm.at[b], dec_v)
        pltpu.sync_copy(y_hbm.at[b], y_v)
        pltpu.sync_copy(il_hbm, il_v.at[pl.ds(0, N)])
        pltpu.sync_copy(tl_hbm, tl_v.at[pl.ds(0, N)])

        iot = lax.iota(jnp.int32, 16)
        zero16 = jnp.zeros((16,), jnp.int32)
        zf16 = jnp.zeros((16,), jnp.float32)
        neg16 = jnp.full((16,), NEG, jnp.float32)

        b16 = jnp.full((16,), b, jnp.int32)
        t_len = jnp.max(plsc.load_gather(il_v, [b16]))
        u_len = jnp.max(plsc.load_gather(tl_v, [b16]))
        d_target = t_len - 1 + u_len
        fin = (plsc.load_gather(enc_v, [jnp.full((16,), t_len - 1, jnp.int32), zero16])
               + plsc.load_gather(dec_v, [jnp.full((16,), u_len, jnp.int32), zero16]))

        for k in range(EBP // 16):
            ebp[pl.ds(16 * k, 16)] = zf16
        for k in range(TMAX // 16):
            t1 = iot + 16 * k
            vals = plsc.load_gather(enc_v, [t1, zero16])
            plsc.store_scatter(ebp, [jnp.full((16,), 574, jnp.int32) - t1], vals)

        for k in range(LUT // 16):
            delk = (iot + 16 * k).astype(jnp.float32) * (1.0 / 64.0)
            t0_v[pl.ds(16 * k, 16)] = _log1p_poly(jnp.exp(-delk))
        for k in range(LUT // 16 - 1):
            lo = t0_v[pl.ds(16 * k, 16)]
            hi = t0_v[pl.ds(16 * k + 1, 16)]
            t1_v[pl.ds(16 * k, 16)] = hi - lo
        t1_v[pl.ds(LUT - 16, 16)] = zf16

        consts = []
        for c in range(NCHUNK):
            u = iot + (16 * c)
            um1 = jnp.clip(u - 1, 0, UMAX - 1)
            yh = plsc.load_gather(y_v, [um1])
            dh = plsc.load_gather(dec_v, [um1, yh])
            ucl = jnp.minimum(u, UMAX)
            db = plsc.load_gather(dec_v, [ucl, zero16])
            consts.extend((u, yh, dh, db))
        consts = tuple(consts)

        pm1 = jnp.maximum(iot - 1, 0)
        lane0 = iot == 0
        p15 = jnp.full((16,), 15, jnp.int32)

        r_init = [jnp.where(iot == 0, 0.0, NEG)] + [neg16] * (NCHUNK - 1)

        def step(d, rp, cs):
            s0 = 575 - d
            off = d * 80
            C = range(NCHUNK)
            us = [cs[4 * c] for c in C]
            yhs = [cs[4 * c + 1] for c in C]
            dhs = [cs[4 * c + 2] for c in C]
            dbs = [cs[4 * c + 3] for c in C]
            tcls = [jnp.clip(d - us[c], 0, TMAX - 1) for c in C]
            ghes = [plsc.load_gather(enc_v, [tcls[c], yhs[c]]) for c in C]
            gvbs = [ebp[pl.ds(s0 + 16 * c, 16)] for c in C]
            shs = [_take(rp[c], pm1) for c in C]
            b15s = [None] + [_take(rp[c - 1], p15) for c in range(1, NCHUNK)]
            hps = [jnp.where(lane0, NEG if c == 0 else b15s[c], shs[c]) for c in C]
            verts = [rp[c] + gvbs[c] + dbs[c] for c in C]
            horizs = [hps[c] + ghes[c] + dhs[c] for c in C]
            ms = [jnp.maximum(verts[c], horizs[c]) for c in C]
            xs = [jnp.minimum(jnp.abs(verts[c] - horizs[c]) * 64.0, 1023.0)
                  for c in C]
            xis = [xs[c].astype(jnp.int32) for c in C]
            frs = [xs[c] - xis[c].astype(jnp.float32) for c in C]
            g0s = [plsc.load_gather(t0_v, [xis[c]]) for c in C]
            g1s = [plsc.load_gather(t1_v, [xis[c]]) for c in C]
            rn = [ms[c] + (g0s[c] + g1s[c] * frs[c]) for c in C]
            for c in C:
                ah[pl.ds(off + 16 * c, 16)] = rn[c]
            return rn

        def loop_body(i, carry):
            cs = carry[:4 * NCHUNK]
            rp = list(carry[4 * NCHUNK:])
            d = 2 * i + 1
            rp = step(d, rp, cs)
            rp = step(d + 1, rp, cs)
            return cs + tuple(rp)

        carry = lax.fori_loop(0, (NDIAG - 1) // 2, loop_body,
                              consts + tuple(r_init))
        step(NDIAG, list(carry[4 * NCHUNK:]), consts)

        av = plsc.load_gather(ah, [jnp.full((16,), d_target * 80 + u_len, jnp.int32)])
        out_v[pl.ds(0, 16)] = av + fin
        pltpu.sync_copy(out_v, out_hbm.at[b])


@jax.jit
def _rnnt_sc(enc, dec, y, il, tl):
    mesh = plsc.VectorSubcoreMesh(core_axis_name="c", subcore_axis_name="s",
                                  num_cores=2, num_subcores=16)
    f = pl.kernel(
        _body,
        out_type=jax.ShapeDtypeStruct((N, 16), jnp.float32),
        mesh=mesh,
        compiler_params=pltpu.CompilerParams(needs_layout_passes=False),
        scratch_types=[
            pltpu.VMEM((TMAX, V), jnp.float32),
            pltpu.VMEM((UMAX + 1, V), jnp.float32),
            pltpu.VMEM((UMAX,), jnp.int32),
            pltpu.VMEM((16,), jnp.int32),
            pltpu.VMEM((16,), jnp.int32),
            pltpu.VMEM((EBP,), jnp.float32),
            pltpu.VMEM((LUT,), jnp.float32),
            pltpu.VMEM((LUT,), jnp.float32),
            pltpu.VMEM((AH,), jnp.float32),
            pltpu.VMEM((16,), jnp.float32),
        ],
    )
    return f(enc, dec, y, il, tl)


def kernel(encoder_out, decoder_out, targets, input_lengths, target_lengths):
    y = targets.astype(jnp.int32)
    il = input_lengths.astype(jnp.int32)
    tl = target_lengths.astype(jnp.int32)
    out = _rnnt_sc(encoder_out, decoder_out, y, il, tl)
    return out[:, 0]


# speedup vs baseline: 60.8824x; 1.0206x over previous
import jax
import jax.numpy as jnp
from jax import lax
from jax.experimental import pallas as pl
from jax.experimental.pallas import tpu as pltpu
from jax.experimental.pallas import tpu_sc as plsc

N = 8
TMAX = 512
UMAX = 64
V = 128
NCHUNK = 5
NEG = -1e30
NDIAG = TMAX - 1 + UMAX
HEB = 528
GDP = 80
LUT = 1040
AH = 80 * (NDIAG + 1)


def _log1p_poly(z):
    p = z * (-0.0172077992) + 0.0817256453
    p = p * z + (-0.188780824)
    p = p * z + 0.314589174
    p = p * z + (-0.496977431)
    p = p * z + 0.999792362
    p = p * z + 3.51102136e-06
    return p


_TAKE_DN = lax.GatherDimensionNumbers(
    offset_dims=(), collapsed_slice_dims=(0,), start_index_map=(0,))


def _take(v, idx):
    return lax.gather(v, idx[:, None], _TAKE_DN, slice_sizes=(1,),
                      mode=lax.GatherScatterMode.PROMISE_IN_BOUNDS)


def _body(enc_hbm, dec_hbm, y_hbm, il_hbm, tl_hbm, out_hbm,
          enc_v, dec_v, y_v, il_v, tl_v, hebp, gdp, t0_v, t1_v, ah, out_v):
    cid = lax.axis_index("c")
    sid = lax.axis_index("s")
    wid = sid * 2 + cid

    @pl.when(wid < N)
    def _run():
        b = wid
        pltpu.sync_copy(enc_hbm.at[b], enc_v)
        pltpu.sync_copy(dec_hbm.at[b], dec_v)
        pltpu.sync_copy(y_hbm.at[b], y_v)
        pltpu.sync_copy(il_hbm, il_v.at[pl.ds(0, N)])
        pltpu.sync_copy(tl_hbm, tl_v.at[pl.ds(0, N)])

        iot = lax.iota(jnp.int32, 16)
        zero16 = jnp.zeros((16,), jnp.int32)
        zf16 = jnp.zeros((16,), jnp.float32)
        neg16 = jnp.full((16,), NEG, jnp.float32)
        pm1 = jnp.maximum(iot - 1, 0)
        lane0 = iot == 0
        p15 = jnp.full((16,), 15, jnp.int32)

        b16 = jnp.full((16,), b, jnp.int32)
        t_len = jnp.max(plsc.load_gather(il_v, [b16]))
        u_len = jnp.max(plsc.load_gather(tl_v, [b16]))
        d_target = t_len - 1 + u_len
        fin = (plsc.load_gather(enc_v, [jnp.full((16,), t_len - 1, jnp.int32), zero16])
               + plsc.load_gather(dec_v, [jnp.full((16,), u_len, jnp.int32), zero16]))

        hebp[pl.ds(0, 16)] = zf16
        carr = zf16
        for k in range(TMAX // 16):
            v = plsc.load_gather(enc_v, [iot + 16 * k, zero16])
            cs = carr + plsc.cumsum(v)
            hebp[pl.ds(16 * k + 1, 16)] = cs
            carr = _take(cs, p15)
        gdp[pl.ds(0, 16)] = zf16
        gdp[pl.ds(64, 16)] = zf16
        carr = zf16
        for k in range(UMAX // 16):
            uv = iot + 16 * k
            yv = plsc.load_gather(y_v, [uv])
            v = plsc.load_gather(dec_v, [uv, yv])
            cs = carr + plsc.cumsum(v)
            gdp[pl.ds(16 * k + 1, 16)] = cs
            carr = _take(cs, p15)

        for k in range(LUT // 16):
            delk = (iot + 16 * k).astype(jnp.float32) * (1.0 / 64.0)
            t0_v[pl.ds(16 * k, 16)] = _log1p_poly(jnp.exp(-delk))
        for k in range(LUT // 16 - 1):
            lo = t0_v[pl.ds(16 * k, 16)]
            hi = t0_v[pl.ds(16 * k + 1, 16)]
            t1_v[pl.ds(16 * k, 16)] = hi - lo
        t1_v[pl.ds(LUT - 16, 16)] = zf16

        consts = []
        for c in range(NCHUNK):
            u = iot + (16 * c)
            um1 = jnp.clip(u - 1, 0, UMAX - 1)
            yh = plsc.load_gather(y_v, [um1])
            ucl = jnp.minimum(u, UMAX)
            db = plsc.load_gather(dec_v, [ucl, zero16])
            consts.extend((u, yh, db))
        consts = tuple(consts)

        r_init = [jnp.where(iot == 0, 0.0, NEG)] + [neg16] * (NCHUNK - 1)

        def step(d, rp, cs):
            off = d * 80
            C = range(NCHUNK)
            us = [cs[3 * c] for c in C]
            yhs = [cs[3 * c + 1] for c in C]
            dbs = [cs[3 * c + 2] for c in C]
            tcls = [jnp.clip(d - us[c], 0, TMAX - 1) for c in C]
            ghes = [plsc.load_gather(enc_v, [tcls[c], yhs[c]]) for c in C]
            shs = [_take(rp[c], pm1) for c in C]
            b15s = [None] + [_take(rp[c - 1], p15) for c in range(1, NCHUNK)]
            hps = [jnp.where(lane0, NEG if c == 0 else b15s[c], shs[c]) for c in C]
            verts = [rp[c] + dbs[c] for c in C]
            horizs = [hps[c] + ghes[c] for c in C]
            ms = [jnp.maximum(verts[c], horizs[c]) for c in C]
            xs = [jnp.minimum(jnp.abs(verts[c] - horizs[c]) * 64.0, 1023.0)
                  for c in C]
            xis = [xs[c].astype(jnp.int32) for c in C]
            frs = [xs[c] - xis[c].astype(jnp.float32) for c in C]
            g0s = [plsc.load_gather(t0_v, [xis[c]]) for c in C]
            g1s = [plsc.load_gather(t1_v, [xis[c]]) for c in C]
            rn = [ms[c] + (g0s[c] + g1s[c] * frs[c]) for c in C]
            for c in C:
                ah[pl.ds(off + 16 * c, 16)] = rn[c]
            return rn

        def loop_body(i, carry):
            cs = carry[:3 * NCHUNK]
            rp = list(carry[3 * NCHUNK:])
            d = 2 * i + 1
            rp = step(d, rp, cs)
            rp = step(d + 1, rp, cs)
            return cs + tuple(rp)

        carry = lax.fori_loop(0, (NDIAG - 1) // 2, loop_body,
                              consts + tuple(r_init))
        step(NDIAG, list(carry[3 * NCHUNK:]), consts)

        av = plsc.load_gather(ah, [jnp.full((16,), d_target * 80 + u_len, jnp.int32)])
        he = plsc.load_gather(hebp, [jnp.full((16,), t_len - 1, jnp.int32)])
        gd = plsc.load_gather(gdp, [jnp.full((16,), u_len, jnp.int32)])
        out_v[pl.ds(0, 16)] = av + fin + (he + gd)
        pltpu.sync_copy(out_v, out_hbm.at[b])


@jax.jit
def _rnnt_sc(enc, dec, y, il, tl):
    mesh = plsc.VectorSubcoreMesh(core_axis_name="c", subcore_axis_name="s",
                                  num_cores=2, num_subcores=16)
    f = pl.kernel(
        _body,
        out_type=jax.ShapeDtypeStruct((N, 16), jnp.float32),
        mesh=mesh,
        compiler_params=pltpu.CompilerParams(needs_layout_passes=False),
        scratch_types=[
            pltpu.VMEM((TMAX, V), jnp.float32),
            pltpu.VMEM((UMAX + 1, V), jnp.float32),
            pltpu.VMEM((UMAX,), jnp.int32),
            pltpu.VMEM((16,), jnp.int32),
            pltpu.VMEM((16,), jnp.int32),
            pltpu.VMEM((HEB,), jnp.float32),
            pltpu.VMEM((GDP,), jnp.float32),
            pltpu.VMEM((LUT,), jnp.float32),
            pltpu.VMEM((LUT,), jnp.float32),
            pltpu.VMEM((AH,), jnp.float32),
            pltpu.VMEM((16,), jnp.float32),
        ],
    )
    return f(enc, dec, y, il, tl)


def kernel(encoder_out, decoder_out, targets, input_lengths, target_lengths):
    y = targets.astype(jnp.int32)
    il = input_lengths.astype(jnp.int32)
    tl = target_lengths.astype(jnp.int32)
    out = _rnnt_sc(encoder_out, decoder_out, y, il, tl)
    return out[:, 0]


# speedup vs baseline: 73.5285x; 1.2077x over previous
import jax
import jax.numpy as jnp
from jax import lax
from jax.experimental import pallas as pl
from jax.experimental.pallas import tpu as pltpu
from jax.experimental.pallas import tpu_sc as plsc

N = 8
TMAX = 512
UMAX = 64
V = 128
NCHUNK = 5
NEG = -1e30
NDIAG = TMAX - 1 + UMAX
HEB = 528
GDP = 80
LUT = 4112
AH = 80 * (NDIAG + 1)


def _log1p_poly(z):
    p = z * (-0.0172077992) + 0.0817256453
    p = p * z + (-0.188780824)
    p = p * z + 0.314589174
    p = p * z + (-0.496977431)
    p = p * z + 0.999792362
    p = p * z + 3.51102136e-06
    return p


_TAKE_DN = lax.GatherDimensionNumbers(
    offset_dims=(), collapsed_slice_dims=(0,), start_index_map=(0,))


def _take(v, idx):
    return lax.gather(v, idx[:, None], _TAKE_DN, slice_sizes=(1,),
                      mode=lax.GatherScatterMode.PROMISE_IN_BOUNDS)


def _body(enc_hbm, dec_hbm, y_hbm, il_hbm, tl_hbm, out_hbm,
          enc_v, dec_v, y_v, il_v, tl_v, hebp, gdp, tnn, ah, out_v, sem):
    cid = lax.axis_index("c")
    sid = lax.axis_index("s")
    wid = sid * 2 + cid

    @pl.when(wid < N)
    def _run():
        b = wid
        cps = [pltpu.async_copy(enc_hbm.at[b], enc_v, sem),
               pltpu.async_copy(dec_hbm.at[b], dec_v, sem),
               pltpu.async_copy(y_hbm.at[b], y_v, sem),
               pltpu.async_copy(il_hbm, il_v.at[pl.ds(0, N)], sem),
               pltpu.async_copy(tl_hbm, tl_v.at[pl.ds(0, N)], sem)]
        for cp in cps:
            cp.wait()

        iot = lax.iota(jnp.int32, 16)
        zero16 = jnp.zeros((16,), jnp.int32)
        zf16 = jnp.zeros((16,), jnp.float32)
        neg16 = jnp.full((16,), NEG, jnp.float32)
        pm1 = jnp.maximum(iot - 1, 0)
        lane0 = iot == 0
        p15 = jnp.full((16,), 15, jnp.int32)

        b16 = jnp.full((16,), b, jnp.int32)
        t_len = jnp.max(plsc.load_gather(il_v, [b16]))
        u_len = jnp.max(plsc.load_gather(tl_v, [b16]))
        d_target = t_len - 1 + u_len
        fin = (plsc.load_gather(enc_v, [jnp.full((16,), t_len - 1, jnp.int32), zero16])
               + plsc.load_gather(dec_v, [jnp.full((16,), u_len, jnp.int32), zero16]))

        hebp[pl.ds(0, 16)] = zf16
        carr = zf16
        for k in range(TMAX // 16):
            v = plsc.load_gather(enc_v, [iot + 16 * k, zero16])
            cs = carr + plsc.cumsum(v)
            hebp[pl.ds(16 * k + 1, 16)] = cs
            carr = _take(cs, p15)
        gdp[pl.ds(0, 16)] = zf16
        gdp[pl.ds(64, 16)] = zf16
        carr = zf16
        for k in range(UMAX // 16):
            uv = iot + 16 * k
            yv = plsc.load_gather(y_v, [uv])
            v = plsc.load_gather(dec_v, [uv, yv])
            cs = carr + plsc.cumsum(v)
            gdp[pl.ds(16 * k + 1, 16)] = cs
            carr = _take(cs, p15)

        def _build(k, carry):
            delk = ((iot + 16 * k).astype(jnp.float32) + 0.5) * (1.0 / 256.0)
            tnn[pl.ds(k * 16, 16)] = _log1p_poly(jnp.exp(-delk))
            return carry
        lax.fori_loop(0, LUT // 16, _build, jnp.int32(0))

        consts = []
        for c in range(NCHUNK):
            u = iot + (16 * c)
            um1 = jnp.clip(u - 1, 0, UMAX - 1)
            yh = plsc.load_gather(y_v, [um1])
            ucl = jnp.minimum(u, UMAX)
            db = plsc.load_gather(dec_v, [ucl, zero16])
            consts.extend((u, yh, db))
        consts = tuple(consts)

        r_init = [jnp.where(iot == 0, 0.0, NEG)] + [neg16] * (NCHUNK - 1)

        def step(d, rp, cs):
            off = d * 80
            C = range(NCHUNK)
            us = [cs[3 * c] for c in C]
            yhs = [cs[3 * c + 1] for c in C]
            dbs = [cs[3 * c + 2] for c in C]
            tcls = [jnp.clip(d - us[c], 0, TMAX - 1) for c in C]
            ghes = [plsc.load_gather(enc_v, [tcls[c], yhs[c]]) for c in C]
            shs = [_take(rp[c], pm1) for c in C]
            b15s = [None] + [_take(rp[c - 1], p15) for c in range(1, NCHUNK)]
            hps = [jnp.where(lane0, NEG if c == 0 else b15s[c], shs[c]) for c in C]
            verts = [rp[c] + dbs[c] for c in C]
            horizs = [hps[c] + ghes[c] for c in C]
            ms = [jnp.maximum(verts[c], horizs[c]) for c in C]
            xs = [jnp.minimum(jnp.abs(verts[c] - horizs[c]) * 256.0, 4095.0)
                  for c in C]
            xis = [xs[c].astype(jnp.int32) for c in C]
            gs = [plsc.load_gather(tnn, [xis[c]]) for c in C]
            rn = [ms[c] + gs[c] for c in C]
            for c in C:
                ah[pl.ds(off + 16 * c, 16)] = rn[c]
            return rn

        def loop_body(i, carry):
            cs = carry[:3 * NCHUNK]
            rp = list(carry[3 * NCHUNK:])
            d = 2 * i + 1
            rp = step(d, rp, cs)
            rp = step(d + 1, rp, cs)
            return cs + tuple(rp)

        carry = lax.fori_loop(0, (NDIAG - 1) // 2, loop_body,
                              consts + tuple(r_init))
        step(NDIAG, list(carry[3 * NCHUNK:]), consts)

        av = plsc.load_gather(ah, [jnp.full((16,), d_target * 80 + u_len, jnp.int32)])
        he = plsc.load_gather(hebp, [jnp.full((16,), t_len - 1, jnp.int32)])
        gd = plsc.load_gather(gdp, [jnp.full((16,), u_len, jnp.int32)])
        out_v[pl.ds(0, 16)] = av + fin + (he + gd)
        pltpu.sync_copy(out_v, out_hbm.at[b])


@jax.jit
def _rnnt_sc(enc, dec, y, il, tl):
    mesh = plsc.VectorSubcoreMesh(core_axis_name="c", subcore_axis_name="s",
                                  num_cores=2, num_subcores=16)
    f = pl.kernel(
        _body,
        out_type=jax.ShapeDtypeStruct((N, 16), jnp.float32),
        mesh=mesh,
        compiler_params=pltpu.CompilerParams(needs_layout_passes=False),
        scratch_types=[
            pltpu.VMEM((TMAX, V), jnp.float32),
            pltpu.VMEM((UMAX + 1, V), jnp.float32),
            pltpu.VMEM((UMAX,), jnp.int32),
            pltpu.VMEM((16,), jnp.int32),
            pltpu.VMEM((16,), jnp.int32),
            pltpu.VMEM((HEB,), jnp.float32),
            pltpu.VMEM((GDP,), jnp.float32),
            pltpu.VMEM((LUT,), jnp.float32),
            pltpu.VMEM((AH,), jnp.float32),
            pltpu.VMEM((16,), jnp.float32),
            pltpu.SemaphoreType.DMA,
        ],
    )
    return f(enc, dec, y, il, tl)


def kernel(encoder_out, decoder_out, targets, input_lengths, target_lengths):
    y = targets.astype(jnp.int32)
    il = input_lengths.astype(jnp.int32)
    tl = target_lengths.astype(jnp.int32)
    out = _rnnt_sc(encoder_out, decoder_out, y, il, tl)
    return out[:, 0]
